# baseline (speedup 1.0000x reference)
import functools

import numpy as np

import jax
import jax.numpy as jnp
from jax import lax
from jax.experimental import pallas as pl
from jax.experimental.pallas import tpu as pltpu
from jax.experimental.pallas import tpu_sc as plsc

B = 16384
NC, NS = 2, 16
NW = NC * NS
BPW = B // NW
CHUNK = 128
NCHUNK = BPW // CHUNK

D_S, D_M, D_E = 6, 54, 6
D_OUT = D_S + D_M + D_E
P_S, P_M, P_E = 8, 56, 8
RB = 8

_mesh = plsc.VectorSubcoreMesh(
    core_axis_name="c", subcore_axis_name="s", num_cores=NC, num_subcores=NS
)


_MAGIC = {6: 10923, 54: 1214}


def _patterns(iota, width, col_off):
    out = []
    for v in range((RB * width) // 16):
        q = v * 16 + iota
        b = (q * _MAGIC[width]) >> 16
        j = q - b * width
        d = b * D_OUT + col_off + j
        out.append((b, j, d))
    return out


_SCRATCH = [
    pltpu.VMEM((NCHUNK, CHUNK), jnp.int32),
    pltpu.VMEM((NCHUNK, CHUNK), jnp.int32),
    pltpu.VMEM((NCHUNK, CHUNK), jnp.int32),
    pltpu.VMEM((BPW, P_S), jnp.float32),
    pltpu.VMEM((BPW, P_M), jnp.float32),
    pltpu.VMEM((BPW, P_E), jnp.float32),
    pltpu.VMEM((BPW * D_OUT,), jnp.float32),
    pltpu.SemaphoreType.DMA,
]


def _se3_body(s_hbm, m_hbm, e_hbm, is_hbm, im_hbm, ie_hbm, out_hbm,
              is_v, im_v, ie_v, s_v, m_v, e_v, out_v, sem):
    wid = lax.axis_index("s") * NC + lax.axis_index("c")
    base = wid * BPW

    for j in range(NCHUNK):
        off = base + j * CHUNK
        pltpu.sync_copy(is_hbm.at[pl.ds(off, CHUNK)], is_v.at[j])
        pltpu.sync_copy(im_hbm.at[pl.ds(off, CHUNK)], im_v.at[j])
        pltpu.sync_copy(ie_hbm.at[pl.ds(off, CHUNK)], ie_v.at[j])

    copies = []
    for j in range(NCHUNK):
        rows = pl.ds(j * CHUNK, CHUNK)
        copies.append(pltpu.async_copy(s_hbm.at[is_v.at[j]], s_v.at[rows], sem))
        copies.append(pltpu.async_copy(m_hbm.at[im_v.at[j]], m_v.at[rows], sem))
        copies.append(pltpu.async_copy(e_hbm.at[ie_v.at[j]], e_v.at[rows], sem))
    for c in copies:
        c.wait()

    iota = lax.iota(jnp.int32, 16)
    tables = [(s_v, D_S, _patterns(iota, D_S, 0)),
              (m_v, D_M, _patterns(iota, D_M, D_S)),
              (e_v, D_E, _patterns(iota, D_E, D_S + D_M))]

    def body(blk, carry):
        b0 = blk * RB
        d0 = b0 * D_OUT
        for src_v, width, pats in tables:
            for bc, jc, dc in pats:
                vec = plsc.load_gather(src_v, [b0 + bc, jc])
                plsc.store_scatter(out_v, [d0 + dc], vec)
        return carry

    lax.fori_loop(0, BPW // RB, body, 0)

    pltpu.sync_copy(out_v, out_hbm.at[pl.ds(base * D_OUT, BPW * D_OUT)])


_se3_lookup = pl.kernel(
    _se3_body,
    out_type=jax.ShapeDtypeStruct((B * D_OUT,), jnp.float32),
    mesh=_mesh,
    compiler_params=pltpu.CompilerParams(
        use_tc_tiling_on_sc=False, needs_layout_passes=False),
    scratch_types=_SCRATCH,
)


def kernel(start_table, mid_table, end_table, idx_start, idx_mid, idx_end):
    flat = _se3_lookup(
        jnp.pad(start_table, ((0, 0), (0, P_S - D_S))),
        jnp.pad(mid_table, ((0, 0), (0, P_M - D_M))),
        jnp.pad(end_table, ((0, 0), (0, P_E - D_E))),
        idx_start.astype(jnp.int32), idx_mid.astype(jnp.int32),
        idx_end.astype(jnp.int32),
    )
    return flat.reshape(B, D_OUT)


# speedup vs baseline: 1.3950x; 1.3950x over previous
import jax
import jax.numpy as jnp
from jax import lax
from jax.experimental import pallas as pl
from jax.experimental.pallas import tpu as pltpu
from jax.experimental.pallas import tpu_sc as plsc

B = 16384
NC, NS = 2, 16
NW = NC * NS
BPW = B // NW
CHUNK = 128
NCHUNK = BPW // CHUNK

D_S, D_M, D_E = 6, 54, 6
D_OUT = D_S + D_M + D_E
P_S, P_M, P_E = 8, 56, 8
RB = 8

_mesh = plsc.VectorSubcoreMesh(
    core_axis_name="c", subcore_axis_name="s", num_cores=NC, num_subcores=NS
)

_MAGIC = {6: 10923, 54: 1214}


def _patterns(iota, width, col_off):
    out = []
    for v in range((RB * width) // 16):
        q = v * 16 + iota
        b = (q * _MAGIC[width]) >> 16
        j = q - b * width
        d = b * D_OUT + col_off + j
        out.append((b, j, d))
    return out


_SCRATCH = [
    pltpu.VMEM((NCHUNK, CHUNK), jnp.int32),
    pltpu.VMEM((NCHUNK, CHUNK), jnp.int32),
    pltpu.VMEM((NCHUNK, CHUNK), jnp.int32),
    pltpu.VMEM((BPW, P_S), jnp.float32),
    pltpu.VMEM((BPW, P_M), jnp.float32),
    pltpu.VMEM((BPW, P_E), jnp.float32),
    pltpu.VMEM((BPW * D_OUT,), jnp.float32),
    pltpu.SemaphoreType.DMA,
]


def _se3_body(s_hbm, m_hbm, e_hbm, is_hbm, im_hbm, ie_hbm, out_hbm,
              is_v, im_v, ie_v, s_v, m_v, e_v, out_v, sem):
    wid = lax.axis_index("s") * NC + lax.axis_index("c")
    base = wid * BPW

    for j in range(NCHUNK):
        off = base + j * CHUNK
        pltpu.sync_copy(is_hbm.at[pl.ds(off, CHUNK)], is_v.at[j])
        pltpu.sync_copy(im_hbm.at[pl.ds(off, CHUNK)], im_v.at[j])
        pltpu.sync_copy(ie_hbm.at[pl.ds(off, CHUNK)], ie_v.at[j])

    copies = []
    for j in range(NCHUNK):
        rows = pl.ds(j * CHUNK, CHUNK)
        copies.append(pltpu.async_copy(s_hbm.at[is_v.at[j]], s_v.at[rows], sem))
        copies.append(pltpu.async_copy(m_hbm.at[im_v.at[j]], m_v.at[rows], sem))
        copies.append(pltpu.async_copy(e_hbm.at[ie_v.at[j]], e_v.at[rows], sem))
    for c in copies:
        c.wait()

    iota = lax.iota(jnp.int32, 16)
    tables = [(s_v, _patterns(iota, D_S, 0)),
              (m_v, _patterns(iota, D_M, D_S)),
              (e_v, _patterns(iota, D_E, D_S + D_M))]

    def body(blk, carry):
        b0 = blk * RB
        d0 = b0 * D_OUT
        for src_v, pats in tables:
            for bc, jc, dc in pats:
                vec = plsc.load_gather(src_v, [b0 + bc, jc])
                plsc.store_scatter(out_v, [d0 + dc], vec)
        return carry

    lax.fori_loop(0, BPW // RB, body, 0)

    pltpu.sync_copy(out_v, out_hbm.at[pl.ds(base * D_OUT, BPW * D_OUT)])


_se3_lookup = pl.kernel(
    _se3_body,
    out_type=jax.ShapeDtypeStruct((B * D_OUT,), jnp.float32),
    mesh=_mesh,
    compiler_params=pltpu.CompilerParams(
        use_tc_tiling_on_sc=False, needs_layout_passes=False),
    scratch_types=_SCRATCH,
)


def _widen(t, extra):
    return jnp.concatenate([t, t[:, :extra]], axis=1)


def kernel(start_table, mid_table, end_table, idx_start, idx_mid, idx_end):
    flat = _se3_lookup(
        _widen(start_table, P_S - D_S),
        _widen(mid_table, P_M - D_M),
        _widen(end_table, P_E - D_E),
        idx_start.astype(jnp.int32), idx_mid.astype(jnp.int32),
        idx_end.astype(jnp.int32),
    )
    return flat.reshape(B, D_OUT)
